# baseline (speedup 1.0000x reference)
import jax
import jax.numpy as jnp
from jax import lax
from jax.experimental import pallas as pl
from jax.experimental.pallas import tpu as pltpu
from jax.experimental.pallas import tpu_sc as plsc

_N, _C, _H, _W = 4, 96, 384, 384
_NC, _NS = 2, 16
_NWORK = _NC * _NS
_HS = _H // 8
_HH = _HS // 2
_LB = _W // 16


def _sc_body(mask_hbm, pred_hbm, tgt_hbm, out_hbm,
             mask_v, pb0, tb0, pb1, tb1, acc_s, acc_a, res_v,
             sp0, st0, sp1, st1):
    wid = lax.axis_index("s") * _NC + lax.axis_index("c")
    n = wid // 8
    h_base = (wid % 8) * _HS

    zero = jnp.zeros((16,), jnp.float32)
    rs, ra, rm = zero, zero, zero

    for half in range(2):
        h0 = h_base + half * _HH
        pltpu.sync_copy(mask_hbm.at[n, 0, pl.ds(h0, _HH), :], mask_v)

        def _zero_row(r, _):
            for l in range(_LB):
                sl = pl.ds(l * 16, 16)
                acc_s[r, sl] = zero
                acc_a[r, sl] = zero
            return 0
        lax.fori_loop(0, _HH, _zero_row, 0)

        def _start(c, pb, tb, sp, st):
            pltpu.make_async_copy(
                pred_hbm.at[n, c, pl.ds(h0, _HH), :], pb, sp).start()
            pltpu.make_async_copy(
                tgt_hbm.at[n, c, pl.ds(h0, _HH), :], tb, st).start()

        def _wait(c, pb, tb, sp, st):
            pltpu.make_async_copy(
                pred_hbm.at[n, c, pl.ds(h0, _HH), :], pb, sp).wait()
            pltpu.make_async_copy(
                tgt_hbm.at[n, c, pl.ds(h0, _HH), :], tb, st).wait()

        def _process(pb, tb):
            def _row(r, _):
                for l in range(_LB):
                    sl = pl.ds(l * 16, 16)
                    p = pb[r, sl]
                    t = tb[r, sl]
                    ad = jnp.abs(p - t)
                    clip = jnp.minimum(ad, 1.0)
                    sm = 0.5 * clip * clip + (ad - clip)
                    plsc.addupdate(acc_s.at[r, sl], sm)
                    plsc.addupdate(acc_a.at[r, sl], ad)
                return 0
            lax.fori_loop(0, _HH, _row, 0)

        _start(0, pb0, tb0, sp0, st0)

        def _cstep(k, _):
            c0 = 2 * k
            _start(c0 + 1, pb1, tb1, sp1, st1)
            _wait(c0, pb0, tb0, sp0, st0)
            _process(pb0, tb0)

            @pl.when(c0 + 2 < _C)
            def _pf():
                _start(c0 + 2, pb0, tb0, sp0, st0)

            _wait(c0 + 1, pb1, tb1, sp1, st1)
            _process(pb1, tb1)
            return 0
        lax.fori_loop(0, _C // 2, _cstep, 0)

        def _red(r, carry):
            crs, cra, crm = carry
            for l in range(_LB):
                sl = pl.ds(l * 16, 16)
                m = mask_v[r, sl]
                crs = crs + acc_s[r, sl] * m
                cra = cra + acc_a[r, sl] * m
                crm = crm + m
            return (crs, cra, crm)
        rs, ra, rm = lax.fori_loop(0, _HH, _red, (rs, ra, rm))

    res_v[0, :] = rs
    res_v[1, :] = ra
    res_v[2, :] = rm
    pltpu.sync_copy(res_v, out_hbm.at[wid])


@jax.jit
def kernel(pred, target, front_position):
    maskf = front_position.astype(jnp.float32)
    buf = lambda: pltpu.VMEM((_HH, _W), jnp.float32)
    run = pl.kernel(
        _sc_body,
        out_type=jax.ShapeDtypeStruct((_NWORK, 3, 16), jnp.float32),
        mesh=plsc.VectorSubcoreMesh(core_axis_name="c", subcore_axis_name="s"),
        scratch_types=[
            buf(), buf(), buf(), buf(), buf(), buf(), buf(),
            pltpu.VMEM((3, 16), jnp.float32),
            pltpu.SemaphoreType.DMA, pltpu.SemaphoreType.DMA,
            pltpu.SemaphoreType.DMA, pltpu.SemaphoreType.DMA,
        ],
    )
    out = run(maskf, pred, target)
    s_tot = jnp.sum(out[:, 0, :])
    a_tot = jnp.sum(out[:, 1, :])
    cnt = jnp.sum(out[:, 2, :]) * _C
    return (s_tot / cnt, a_tot / cnt)


# speedup vs baseline: 1.7844x; 1.7844x over previous
import jax
import jax.numpy as jnp
from jax import lax
from jax.experimental import pallas as pl
from jax.experimental.pallas import tpu as pltpu
from jax.experimental.pallas import tpu_sc as plsc

_N, _C, _H, _W = 4, 96, 384, 384
_CT = 64
_CS = _C - _CT
_J = 6
_HB = _H // _J

_NC, _NS = 2, 16
_NWORK = _NC * _NS
_HS = _H // 8
_HH = _HS // 2
_LB = _W // 16


def _tc_body(mask_ref, pred_ref, tgt_ref, s_ref, a_ref, c_ref):
    n = pl.program_id(0)
    j = pl.program_id(1)

    @pl.when(jnp.logical_and(n == 0, j == 0))
    def _init():
        s_ref[0, 0] = 0.0
        a_ref[0, 0] = 0.0
        c_ref[0, 0] = 0.0

    m = mask_ref[0, 0]
    acc_s = jnp.zeros((_HB, _W), jnp.float32)
    acc_a = jnp.zeros((_HB, _W), jnp.float32)
    for c in range(_CT):
        p = pred_ref[0, c]
        t = tgt_ref[0, c]
        ad = jnp.abs(p - t) * m
        clip = jnp.minimum(ad, 1.0)
        sm = 0.5 * clip * clip + (ad - clip)
        acc_s = acc_s + sm
        acc_a = acc_a + ad

    s_ref[0, 0] += jnp.sum(acc_s)
    a_ref[0, 0] += jnp.sum(acc_a)
    c_ref[0, 0] += jnp.sum(m)


def _sc_body(mask_hbm, pred_hbm, tgt_hbm, out_hbm,
             mask_v, pb0, tb0, pb1, tb1, acc_s, acc_a, res_v,
             sp0, st0, sp1, st1):
    wid = lax.axis_index("s") * _NC + lax.axis_index("c")
    n = wid // 8
    h_base = (wid % 8) * _HS

    zero = jnp.zeros((16,), jnp.float32)
    rs, ra = zero, zero

    for half in range(2):
        h0 = h_base + half * _HH
        pltpu.sync_copy(mask_hbm.at[n, 0, pl.ds(h0, _HH), :], mask_v)

        def _zero_row(r, _):
            for l in range(_LB):
                sl = pl.ds(l * 16, 16)
                acc_s[r, sl] = zero
                acc_a[r, sl] = zero
            return 0
        lax.fori_loop(0, _HH, _zero_row, 0)

        def _start(c, pb, tb, sp, st):
            pltpu.make_async_copy(
                pred_hbm.at[n, c, pl.ds(h0, _HH), :], pb, sp).start()
            pltpu.make_async_copy(
                tgt_hbm.at[n, c, pl.ds(h0, _HH), :], tb, st).start()

        def _wait(c, pb, tb, sp, st):
            pltpu.make_async_copy(
                pred_hbm.at[n, c, pl.ds(h0, _HH), :], pb, sp).wait()
            pltpu.make_async_copy(
                tgt_hbm.at[n, c, pl.ds(h0, _HH), :], tb, st).wait()

        def _process(pb, tb):
            def _row(r, _):
                for l in range(_LB):
                    sl = pl.ds(l * 16, 16)
                    p = pb[r, sl]
                    t = tb[r, sl]
                    ad = jnp.abs(p - t)
                    clip = jnp.minimum(ad, 1.0)
                    sm = 0.5 * clip * clip + (ad - clip)
                    plsc.addupdate(acc_s.at[r, sl], sm)
                    plsc.addupdate(acc_a.at[r, sl], ad)
                return 0
            lax.fori_loop(0, _HH, _row, 0)

        _start(_CT, pb0, tb0, sp0, st0)

        def _cstep(k, _):
            c0 = _CT + 2 * k
            _start(c0 + 1, pb1, tb1, sp1, st1)
            _wait(c0, pb0, tb0, sp0, st0)
            _process(pb0, tb0)

            @pl.when(c0 + 2 < _C)
            def _pf():
                _start(c0 + 2, pb0, tb0, sp0, st0)

            _wait(c0 + 1, pb1, tb1, sp1, st1)
            _process(pb1, tb1)
            return 0
        lax.fori_loop(0, _CS // 2, _cstep, 0)

        def _red(r, carry):
            crs, cra = carry
            for l in range(_LB):
                sl = pl.ds(l * 16, 16)
                m = mask_v[r, sl]
                crs = crs + acc_s[r, sl] * m
                cra = cra + acc_a[r, sl] * m
            return (crs, cra)
        rs, ra = lax.fori_loop(0, _HH, _red, (rs, ra))

    res_v[0, :] = rs
    res_v[1, :] = ra
    pltpu.sync_copy(res_v, out_hbm.at[wid])


@jax.jit
def kernel(pred, target, front_position):
    maskf = front_position.astype(jnp.float32)

    buf = lambda: pltpu.VMEM((_HH, _W), jnp.float32)
    sc_run = pl.kernel(
        _sc_body,
        out_type=jax.ShapeDtypeStruct((_NWORK, 2, 16), jnp.float32),
        mesh=plsc.VectorSubcoreMesh(core_axis_name="c", subcore_axis_name="s"),
        scratch_types=[
            buf(), buf(), buf(), buf(), buf(), buf(), buf(),
            pltpu.VMEM((2, 16), jnp.float32),
            pltpu.SemaphoreType.DMA, pltpu.SemaphoreType.DMA,
            pltpu.SemaphoreType.DMA, pltpu.SemaphoreType.DMA,
        ],
    )
    sc_out = sc_run(maskf, pred, target)

    scal = jax.ShapeDtypeStruct((1, 1), jnp.float32)
    s_tc, a_tc, m_cnt = pl.pallas_call(
        _tc_body,
        grid=(_N, _J),
        compiler_params=pltpu.CompilerParams(
            vmem_limit_bytes=128 * 1024 * 1024),
        in_specs=[
            pl.BlockSpec((1, 1, _HB, _W), lambda n, j: (n, 0, j, 0)),
            pl.BlockSpec((1, _CT, _HB, _W), lambda n, j: (n, 0, j, 0)),
            pl.BlockSpec((1, _CT, _HB, _W), lambda n, j: (n, 0, j, 0)),
        ],
        out_specs=[
            pl.BlockSpec(memory_space=pltpu.SMEM),
            pl.BlockSpec(memory_space=pltpu.SMEM),
            pl.BlockSpec(memory_space=pltpu.SMEM),
        ],
        out_shape=[scal, scal, scal],
    )(maskf, pred, target)

    cnt = m_cnt[0, 0] * _C
    s_tot = s_tc[0, 0] + jnp.sum(sc_out[:, 0, :])
    a_tot = a_tc[0, 0] + jnp.sum(sc_out[:, 1, :])
    return (s_tot / cnt, a_tot / cnt)


# speedup vs baseline: 2.0354x; 1.1406x over previous
import jax
import jax.numpy as jnp
from jax import lax
from jax.experimental import pallas as pl
from jax.experimental.pallas import tpu as pltpu

_N, _C, _H, _W = 4, 96, 384, 384
_CB = 32
_NCB = _C // _CB
_HB = 128
_NHB = _H // _HB
_RT = 16
_NRT = _HB // _RT


def _body(mask_ref, pred_ref, tgt_ref, s_ref, a_ref, c_ref, loss_ref, dm_ref):
    n = pl.program_id(0)
    h = pl.program_id(1)
    c = pl.program_id(2)
    first = jnp.logical_and(n == 0, jnp.logical_and(h == 0, c == 0))

    @pl.when(first)
    def _init():
        s_ref[0, 0] = 0.0
        a_ref[0, 0] = 0.0
        c_ref[0, 0] = 0.0

    acc_s = jnp.zeros((_RT, _W), jnp.float32)
    acc_a = jnp.zeros((_RT, _W), jnp.float32)
    for rr in range(_NRT):
        rows = pl.ds(rr * _RT, _RT)
        m = mask_ref[0, 0, rows, :].astype(jnp.float32)
        for cc in range(_CB):
            p = pred_ref[0, cc, rows, :]
            t = tgt_ref[0, cc, rows, :]
            ad = jnp.abs(p - t) * m
            clip = jnp.minimum(ad, 1.0)
            sm = 0.5 * clip * clip + (ad - clip)
            acc_s = acc_s + sm
            acc_a = acc_a + ad

        @pl.when(c == 0)
        def _cnt():
            c_ref[0, 0] += jnp.sum(m)

    s_ref[0, 0] += jnp.sum(acc_s)
    a_ref[0, 0] += jnp.sum(acc_a)

    last = jnp.logical_and(n == _N - 1,
                           jnp.logical_and(h == _NHB - 1, c == _NCB - 1))

    @pl.when(last)
    def _fin():
        cnt = c_ref[0, 0] * _C
        loss_ref[0, 0] = s_ref[0, 0] / cnt
        dm_ref[0, 0] = a_ref[0, 0] / cnt


@jax.jit
def kernel(pred, target, front_position):
    mask_i8 = front_position.view(jnp.int8)

    scal = jax.ShapeDtypeStruct((1, 1), jnp.float32)
    _, _, _, loss, dm = pl.pallas_call(
        _body,
        grid=(_N, _NHB, _NCB),
        compiler_params=pltpu.CompilerParams(
            vmem_limit_bytes=128 * 1024 * 1024),
        in_specs=[
            pl.BlockSpec((1, 1, _HB, _W), lambda n, h, c: (n, 0, h, 0)),
            pl.BlockSpec((1, _CB, _HB, _W), lambda n, h, c: (n, c, h, 0)),
            pl.BlockSpec((1, _CB, _HB, _W), lambda n, h, c: (n, c, h, 0)),
        ],
        out_specs=[
            pl.BlockSpec(memory_space=pltpu.SMEM),
            pl.BlockSpec(memory_space=pltpu.SMEM),
            pl.BlockSpec(memory_space=pltpu.SMEM),
            pl.BlockSpec(memory_space=pltpu.SMEM),
            pl.BlockSpec(memory_space=pltpu.SMEM),
        ],
        out_shape=[scal, scal, scal, scal, scal],
    )(mask_i8, pred, target)

    return (loss[0, 0], dm[0, 0])


# speedup vs baseline: 2.1175x; 1.0404x over previous
import jax
import jax.numpy as jnp
from jax.experimental import pallas as pl
from jax.experimental.pallas import tpu as pltpu

_N, _C, _H, _W = 4, 96, 384, 384
_J = 6
_HB = _H // _J


def _body(mask_ref, pred_ref, tgt_ref, s_ref, a_ref, c_ref, loss_ref, dm_ref):
    n = pl.program_id(0)
    j = pl.program_id(1)

    @pl.when(jnp.logical_and(n == 0, j == 0))
    def _init():
        s_ref[0, 0] = 0.0
        a_ref[0, 0] = 0.0
        c_ref[0, 0] = 0.0

    m = mask_ref[0, 0]
    acc_s = jnp.zeros((_HB, _W), jnp.float32)
    acc_a = jnp.zeros((_HB, _W), jnp.float32)
    for c in range(_C):
        p = pred_ref[0, c]
        t = tgt_ref[0, c]
        ad = jnp.abs(p - t) * m
        clip = jnp.minimum(ad, 1.0)
        sm = 0.5 * clip * clip + (ad - clip)
        acc_s = acc_s + sm
        acc_a = acc_a + ad

    s_ref[0, 0] += jnp.sum(acc_s)
    a_ref[0, 0] += jnp.sum(acc_a)
    c_ref[0, 0] += jnp.sum(m)

    @pl.when(jnp.logical_and(n == _N - 1, j == _J - 1))
    def _fin():
        cnt = c_ref[0, 0] * _C
        loss_ref[0, 0] = s_ref[0, 0] / cnt
        dm_ref[0, 0] = a_ref[0, 0] / cnt


@jax.jit
def kernel(pred, target, front_position):
    scal = jax.ShapeDtypeStruct((1, 1), jnp.float32)
    _, _, _, loss, dm = pl.pallas_call(
        _body,
        grid=(_N, _J),
        compiler_params=pltpu.CompilerParams(
            vmem_limit_bytes=128 * 1024 * 1024),
        in_specs=[
            pl.BlockSpec((1, 1, _HB, _W), lambda n, j: (n, 0, j, 0)),
            pl.BlockSpec((1, _C, _HB, _W), lambda n, j: (n, 0, j, 0)),
            pl.BlockSpec((1, _C, _HB, _W), lambda n, j: (n, 0, j, 0)),
        ],
        out_specs=[
            pl.BlockSpec(memory_space=pltpu.SMEM),
            pl.BlockSpec(memory_space=pltpu.SMEM),
            pl.BlockSpec(memory_space=pltpu.SMEM),
            pl.BlockSpec(memory_space=pltpu.SMEM),
            pl.BlockSpec(memory_space=pltpu.SMEM),
        ],
        out_shape=[scal, scal, scal, scal, scal],
    )(front_position.astype(jnp.float32), pred, target)

    return (loss[0, 0], dm[0, 0])


# speedup vs baseline: 2.1391x; 1.0102x over previous
import jax
import jax.numpy as jnp
from jax.experimental import pallas as pl
from jax.experimental.pallas import tpu as pltpu

_N, _C, _H, _W = 4, 96, 384, 384
_J = 6
_HB = _H // _J


def _body(mask_ref, pred_ref, tgt_ref, s_ref, a_ref, c_ref, loss_ref, dm_ref):
    n = pl.program_id(0)
    j = pl.program_id(1)

    @pl.when(jnp.logical_and(n == 0, j == 0))
    def _init():
        s_ref[0, 0] = 0.0
        a_ref[0, 0] = 0.0
        c_ref[0, 0] = 0.0

    m = mask_ref[0, 0].astype(jnp.float32)
    acc_s = jnp.zeros((_HB, _W), jnp.float32)
    acc_a = jnp.zeros((_HB, _W), jnp.float32)
    for c in range(_C):
        p = pred_ref[0, c]
        t = tgt_ref[0, c]
        ad = jnp.abs(p - t) * m
        clip = jnp.minimum(ad, 1.0)
        sm = 0.5 * clip * clip + (ad - clip)
        acc_s = acc_s + sm
        acc_a = acc_a + ad

    s_ref[0, 0] += jnp.sum(acc_s)
    a_ref[0, 0] += jnp.sum(acc_a)
    c_ref[0, 0] += jnp.sum(m)

    @pl.when(jnp.logical_and(n == _N - 1, j == _J - 1))
    def _fin():
        cnt = c_ref[0, 0] * _C
        loss_ref[0, 0] = s_ref[0, 0] / cnt
        dm_ref[0, 0] = a_ref[0, 0] / cnt


@jax.jit
def kernel(pred, target, front_position):
    scal = jax.ShapeDtypeStruct((1, 1), jnp.float32)
    _, _, _, loss, dm = pl.pallas_call(
        _body,
        grid=(_N, _J),
        compiler_params=pltpu.CompilerParams(
            vmem_limit_bytes=128 * 1024 * 1024),
        in_specs=[
            pl.BlockSpec((1, 1, _HB, _W), lambda n, j: (n, 0, j, 0)),
            pl.BlockSpec((1, _C, _HB, _W), lambda n, j: (n, 0, j, 0)),
            pl.BlockSpec((1, _C, _HB, _W), lambda n, j: (n, 0, j, 0)),
        ],
        out_specs=[
            pl.BlockSpec(memory_space=pltpu.SMEM),
            pl.BlockSpec(memory_space=pltpu.SMEM),
            pl.BlockSpec(memory_space=pltpu.SMEM),
            pl.BlockSpec(memory_space=pltpu.SMEM),
            pl.BlockSpec(memory_space=pltpu.SMEM),
        ],
        out_shape=[scal, scal, scal, scal, scal],
    )(front_position.view(jnp.int8), pred, target)

    return (loss[0, 0], dm[0, 0])
